# baseline (speedup 1.0000x reference)
import functools

import jax
import jax.numpy as jnp
from jax import lax
from jax.experimental import pallas as pl
from jax.experimental.pallas import tpu as pltpu
from jax.experimental.pallas import tpu_sc as plsc

N = 10000
E = 320000
HID = 32
NW = 32
EPW = E // NW
CH = 128
NG = 80
EPW_PAD = NG * CH
NPAD = N + 112
DUMMY = N
RPS = NPAD // 16

def _deg_body(dst_g, ones_hbm, zeros_hbm, deg_out, idx_v, ones_v, zstage_v, acc):
    cid = lax.axis_index("c")
    sid = lax.axis_index("s")
    wid = sid * 2 + cid
    pltpu.sync_copy(dst_g.at[wid], idx_v)
    pltpu.sync_copy(ones_hbm, ones_v)

    @pl.when(sid == 0)
    def _():
        pltpu.sync_copy(zeros_hbm, zstage_v)
        pltpu.sync_copy(zstage_v, acc)

    plsc.subcore_barrier()

    @pl.loop(0, NG)
    def _(j):
        pltpu.sync_copy(ones_v, acc.at[idx_v.at[j]], add=True)

    plsc.subcore_barrier()

    @pl.when(sid == 0)
    def _():
        pltpu.sync_copy(acc, deg_out.at[cid])


def _agg_body(src_g, dst_g, hp_hbm, zeros_hbm, agg_out,
              sidx_v, didx_v, rows0_v, rows1_v, zbuf_v, acc, sem0, sem1):
    cid = lax.axis_index("c")
    sid = lax.axis_index("s")
    wid = sid * 2 + cid
    pltpu.sync_copy(src_g.at[wid], sidx_v)
    pltpu.sync_copy(dst_g.at[wid], didx_v)
    pltpu.sync_copy(zeros_hbm, zbuf_v)
    pltpu.sync_copy(zbuf_v, acc.at[pl.ds(sid * RPS, RPS)])
    plsc.subcore_barrier()

    @pl.loop(0, NG // 2)
    def _(i):
        g = i * 2
        pltpu.async_copy(hp_hbm.at[sidx_v.at[g]], rows0_v, sem0).wait()
        pltpu.sync_copy(rows0_v, acc.at[didx_v.at[g]], add=True)
        pltpu.async_copy(hp_hbm.at[sidx_v.at[g + 1]], rows1_v, sem1).wait()
        pltpu.sync_copy(rows1_v, acc.at[didx_v.at[g + 1]], add=True)

    plsc.subcore_barrier()
    pltpu.sync_copy(acc.at[pl.ds(sid * RPS, RPS)],
                    agg_out.at[cid, pl.ds(sid * RPS, RPS)])


@functools.cache
def _sc_kernels():
    mesh = plsc.VectorSubcoreMesh(core_axis_name="c", subcore_axis_name="s",
                                  num_cores=2, num_subcores=16)
    params = pltpu.CompilerParams(use_tc_tiling_on_sc=False)
    deg = pl.kernel(
        _deg_body,
        out_type=jax.ShapeDtypeStruct((2, NPAD), jnp.float32),
        mesh=mesh,
        compiler_params=params,
        scratch_types=[
            pltpu.VMEM((NG, CH), jnp.int32),
            pltpu.VMEM((CH,), jnp.float32),
            pltpu.VMEM((NPAD,), jnp.float32),
            pltpu.VMEM_SHARED((NPAD,), jnp.float32),
        ],
    )
    agg = pl.kernel(
        _agg_body,
        out_type=jax.ShapeDtypeStruct((2, NPAD, HID), jnp.float32),
        mesh=mesh,
        compiler_params=params,
        scratch_types=[
            pltpu.VMEM((NG, CH), jnp.int32),
            pltpu.VMEM((NG, CH), jnp.int32),
            pltpu.VMEM((CH, HID), jnp.float32),
            pltpu.VMEM((CH, HID), jnp.float32),
            pltpu.VMEM((RPS, HID), jnp.float32),
            pltpu.VMEM_SHARED((NPAD, HID), jnp.float32),
            pltpu.SemaphoreType.DMA,
            pltpu.SemaphoreType.DMA,
        ],
    )
    return deg, agg


def _enc_body(obs_ref, act_ref, degT_ref, w1a, w1b, b1, w2, b2, wg,
              x_out, hp_out, dinv_out):
    f32 = jnp.float32
    x1 = jnp.dot(obs_ref[...], w1a[...], preferred_element_type=f32)
    x1 = x1 + jnp.dot(act_ref[...], w1b[...], preferred_element_type=f32)
    x1 = jnp.maximum(x1 + b1[...], 0.0)
    x = jnp.maximum(jnp.dot(x1, w2[...], preferred_element_type=f32) + b2[...], 0.0)
    x_out[...] = x
    h = jnp.dot(x, wg[...], preferred_element_type=f32)
    d = degT_ref[...]
    deg = d[:, 0:1] + d[:, 1:2] + 1.0
    dinv = lax.rsqrt(deg)
    dinv_out[...] = dinv
    hp_out[...] = h * dinv


def _enc(obs, act, degT, w1a, w1b, b1, w2, b2, wg):
    return pl.pallas_call(
        _enc_body,
        out_shape=[
            jax.ShapeDtypeStruct((N, HID), jnp.float32),
            jax.ShapeDtypeStruct((N, HID), jnp.float32),
            jax.ShapeDtypeStruct((N, 1), jnp.float32),
        ],
    )(obs, act, degT, w1a, w1b, b1, w2, b2, wg)


def _post_body(a0, a1, hp, dinv, bg, wd, bd, x, wp1a, wp1b, bp1, wp2, bp2, out):
    f32 = jnp.float32
    agg = a0[...] + a1[...] + hp[...]
    xg = jnp.maximum(agg * dinv[...] + bg[...], 0.0)
    xg = jnp.maximum(jnp.dot(xg, wd[...], preferred_element_type=f32) + bd[...], 0.0)
    xp = jnp.dot(xg, wp1a[...], preferred_element_type=f32)
    xp = xp + jnp.dot(x[...], wp1b[...], preferred_element_type=f32)
    xp = jnp.maximum(xp + bp1[...], 0.0)
    xp = jnp.maximum(jnp.dot(xp, wp2[...], preferred_element_type=f32) + bp2[...], 0.0)
    out[...] = xp


def _post(a0, a1, hp, dinv, bg, wd, bd, x, wp1a, wp1b, bp1, wp2, bp2):
    return pl.pallas_call(
        _post_body,
        out_shape=jax.ShapeDtypeStruct((N, HID), jnp.float32),
    )(a0, a1, hp, dinv, bg, wd, bd, x, wp1a, wp1b, bp1, wp2, bp2)


_VBLK = 32000
_VGRID = (N * HID) // _VBLK


def _value_body(vec_ref, wv1_ref, bv1_ref, wv2_ref, bv2_ref, out_ref, acc_ref):
    k = pl.program_id(0)

    @pl.when(k == 0)
    def _():
        acc_ref[...] = jnp.zeros_like(acc_ref)

    acc_ref[...] += jnp.dot(vec_ref[...], wv1_ref[...],
                            preferred_element_type=jnp.float32)

    @pl.when(k == _VGRID - 1)
    def _():
        v = jnp.maximum(acc_ref[...] + bv1_ref[...], 0.0)
        out_ref[...] = jnp.dot(v, wv2_ref[...],
                               preferred_element_type=jnp.float32) + bv2_ref[...]


def _value(vec, wv1, bv1, wv2, bv2):
    return pl.pallas_call(
        _value_body,
        grid=(_VGRID,),
        in_specs=[
            pl.BlockSpec((1, _VBLK), lambda k: (0, k)),
            pl.BlockSpec((_VBLK, 64), lambda k: (k, 0)),
            pl.BlockSpec((1, 64), lambda k: (0, 0)),
            pl.BlockSpec((64, 1), lambda k: (0, 0)),
            pl.BlockSpec((1, 1), lambda k: (0, 0)),
        ],
        out_specs=pl.BlockSpec((1, 1), lambda k: (0, 0)),
        out_shape=jax.ShapeDtypeStruct((1, 1), jnp.float32),
        scratch_shapes=[pltpu.VMEM((1, 64), jnp.float32)],
    )(vec, wv1, bv1, wv2, bv2)


def kernel(observation, action, edge_index, We1, be1, We2, be2, Wg, bg,
           Wd, bd, Wp1, bp1, Wp2, bp2, Wv1, bv1, Wv2, bv2):
    obs = observation.reshape(N, -1)
    act = action.reshape(N, -1)
    ei = edge_index.astype(jnp.int32)
    pad = jnp.full((NW, EPW_PAD - EPW), DUMMY, jnp.int32)
    src_g = jnp.concatenate([ei[0].reshape(NW, EPW), pad], 1).reshape(NW, NG, CH)
    dst_g = jnp.concatenate([ei[1].reshape(NW, EPW), pad], 1).reshape(NW, NG, CH)

    ones128 = jnp.ones((CH,), jnp.float32)
    zeros_n = jnp.zeros((NPAD,), jnp.float32)
    zeros_rows = jnp.zeros((RPS, HID), jnp.float32)

    _deg, _agg = _sc_kernels()
    deg2 = _deg(dst_g, ones128, zeros_n)
    degT = jnp.transpose(deg2[:, :N])
    X, hp, dinv = _enc(obs, act, degT, We1[:128], We1[128:],
                       be1.reshape(1, -1), We2, be2.reshape(1, -1), Wg)
    hp_pad = jnp.concatenate([hp, jnp.zeros((NPAD - N, HID), jnp.float32)], 0)

    agg2 = _agg(src_g, dst_g, hp_pad, zeros_rows)

    xp2 = _post(agg2[0, :N], agg2[1, :N], hp, dinv, bg.reshape(1, -1),
                Wd, bd.reshape(1, -1), X, Wp1[:HID], Wp1[HID:],
                bp1.reshape(1, -1), Wp2, bp2.reshape(1, -1))
    vec = xp2.reshape(1, N * HID)
    out = _value(vec, Wv1, bv1.reshape(1, -1), Wv2, bv2.reshape(1, 1))
    return out.reshape(1)


# speedup vs baseline: 1.0004x; 1.0004x over previous
import functools

import jax
import jax.numpy as jnp
from jax import lax
from jax.experimental import pallas as pl
from jax.experimental.pallas import tpu as pltpu
from jax.experimental.pallas import tpu_sc as plsc

N = 10000
E = 320000
HID = 32
NW = 32
EPW = E // NW
CH = 128
NG = 80
EPW_PAD = NG * CH
NPAD = N + 112
DUMMY = N
RPS = NPAD // 16

def _deg_body(dst_g, ones_hbm, zeros_hbm, deg_out, idx_v, ones_v, zstage_v, acc):
    cid = lax.axis_index("c")
    sid = lax.axis_index("s")
    wid = sid * 2 + cid
    pltpu.sync_copy(dst_g.at[wid], idx_v)
    pltpu.sync_copy(ones_hbm, ones_v)

    @pl.when(sid == 0)
    def _():
        pltpu.sync_copy(zeros_hbm, zstage_v)
        pltpu.sync_copy(zstage_v, acc)

    plsc.subcore_barrier()

    @pl.loop(0, NG)
    def _(j):
        pltpu.sync_copy(ones_v, acc.at[idx_v.at[j]], add=True)

    plsc.subcore_barrier()

    @pl.when(sid == 0)
    def _():
        pltpu.sync_copy(acc, deg_out.at[cid])


def _agg_body(src_g, dst_g, hp_hbm, zeros_hbm, agg_out,
              sidx_v, didx_v, rows0_v, rows1_v, zbuf_v, acc, sem0, sem1):
    cid = lax.axis_index("c")
    sid = lax.axis_index("s")
    wid = sid * 2 + cid
    pltpu.sync_copy(src_g.at[wid], sidx_v)
    pltpu.sync_copy(dst_g.at[wid], didx_v)
    pltpu.sync_copy(zeros_hbm, zbuf_v)
    pltpu.sync_copy(zbuf_v, acc.at[pl.ds(sid * RPS, RPS)])
    plsc.subcore_barrier()

    @pl.loop(0, NG)
    def _(j):
        pltpu.async_copy(hp_hbm.at[sidx_v.at[j]], rows0_v, sem0).wait()
        pltpu.sync_copy(rows0_v, acc.at[didx_v.at[j]], add=True)

    plsc.subcore_barrier()
    pltpu.sync_copy(acc.at[pl.ds(sid * RPS, RPS)],
                    agg_out.at[cid, pl.ds(sid * RPS, RPS)])


@functools.cache
def _sc_kernels():
    mesh = plsc.VectorSubcoreMesh(core_axis_name="c", subcore_axis_name="s",
                                  num_cores=2, num_subcores=16)
    params = pltpu.CompilerParams(use_tc_tiling_on_sc=False)
    deg = pl.kernel(
        _deg_body,
        out_type=jax.ShapeDtypeStruct((2, NPAD), jnp.float32),
        mesh=mesh,
        compiler_params=params,
        scratch_types=[
            pltpu.VMEM((NG, CH), jnp.int32),
            pltpu.VMEM((CH,), jnp.float32),
            pltpu.VMEM((NPAD,), jnp.float32),
            pltpu.VMEM_SHARED((NPAD,), jnp.float32),
        ],
    )
    agg = pl.kernel(
        _agg_body,
        out_type=jax.ShapeDtypeStruct((2, NPAD, HID), jnp.float32),
        mesh=mesh,
        compiler_params=params,
        scratch_types=[
            pltpu.VMEM((NG, CH), jnp.int32),
            pltpu.VMEM((NG, CH), jnp.int32),
            pltpu.VMEM((CH, HID), jnp.float32),
            pltpu.VMEM((CH, HID), jnp.float32),
            pltpu.VMEM((RPS, HID), jnp.float32),
            pltpu.VMEM_SHARED((NPAD, HID), jnp.float32),
            pltpu.SemaphoreType.DMA,
            pltpu.SemaphoreType.DMA,
        ],
    )
    return deg, agg


def _enc_body(obs_ref, act_ref, degT_ref, w1a, w1b, b1, w2, b2, wg,
              x_out, hp_out, dinv_out):
    f32 = jnp.float32
    x1 = jnp.dot(obs_ref[...], w1a[...], preferred_element_type=f32)
    x1 = x1 + jnp.dot(act_ref[...], w1b[...], preferred_element_type=f32)
    x1 = jnp.maximum(x1 + b1[...], 0.0)
    x = jnp.maximum(jnp.dot(x1, w2[...], preferred_element_type=f32) + b2[...], 0.0)
    x_out[...] = x
    h = jnp.dot(x, wg[...], preferred_element_type=f32)
    d = degT_ref[...]
    deg = d[:, 0:1] + d[:, 1:2] + 1.0
    dinv = lax.rsqrt(deg)
    dinv_out[...] = dinv
    hp_out[...] = h * dinv


def _enc(obs, act, degT, w1a, w1b, b1, w2, b2, wg):
    return pl.pallas_call(
        _enc_body,
        out_shape=[
            jax.ShapeDtypeStruct((N, HID), jnp.float32),
            jax.ShapeDtypeStruct((N, HID), jnp.float32),
            jax.ShapeDtypeStruct((N, 1), jnp.float32),
        ],
    )(obs, act, degT, w1a, w1b, b1, w2, b2, wg)


def _post_body(a0, a1, hp, dinv, bg, wd, bd, x, wp1a, wp1b, bp1, wp2, bp2, out):
    f32 = jnp.float32
    agg = a0[...] + a1[...] + hp[...]
    xg = jnp.maximum(agg * dinv[...] + bg[...], 0.0)
    xg = jnp.maximum(jnp.dot(xg, wd[...], preferred_element_type=f32) + bd[...], 0.0)
    xp = jnp.dot(xg, wp1a[...], preferred_element_type=f32)
    xp = xp + jnp.dot(x[...], wp1b[...], preferred_element_type=f32)
    xp = jnp.maximum(xp + bp1[...], 0.0)
    xp = jnp.maximum(jnp.dot(xp, wp2[...], preferred_element_type=f32) + bp2[...], 0.0)
    out[...] = xp


def _post(a0, a1, hp, dinv, bg, wd, bd, x, wp1a, wp1b, bp1, wp2, bp2):
    return pl.pallas_call(
        _post_body,
        out_shape=jax.ShapeDtypeStruct((N, HID), jnp.float32),
    )(a0, a1, hp, dinv, bg, wd, bd, x, wp1a, wp1b, bp1, wp2, bp2)


_VBLK = 32000
_VGRID = (N * HID) // _VBLK


def _value_body(vec_ref, wv1_ref, bv1_ref, wv2_ref, bv2_ref, out_ref, acc_ref):
    k = pl.program_id(0)

    @pl.when(k == 0)
    def _():
        acc_ref[...] = jnp.zeros_like(acc_ref)

    acc_ref[...] += jnp.dot(vec_ref[...], wv1_ref[...],
                            preferred_element_type=jnp.float32)

    @pl.when(k == _VGRID - 1)
    def _():
        v = jnp.maximum(acc_ref[...] + bv1_ref[...], 0.0)
        out_ref[...] = jnp.dot(v, wv2_ref[...],
                               preferred_element_type=jnp.float32) + bv2_ref[...]


def _value(vec, wv1, bv1, wv2, bv2):
    return pl.pallas_call(
        _value_body,
        grid=(_VGRID,),
        in_specs=[
            pl.BlockSpec((1, _VBLK), lambda k: (0, k)),
            pl.BlockSpec((_VBLK, 64), lambda k: (k, 0)),
            pl.BlockSpec((1, 64), lambda k: (0, 0)),
            pl.BlockSpec((64, 1), lambda k: (0, 0)),
            pl.BlockSpec((1, 1), lambda k: (0, 0)),
        ],
        out_specs=pl.BlockSpec((1, 1), lambda k: (0, 0)),
        out_shape=jax.ShapeDtypeStruct((1, 1), jnp.float32),
        scratch_shapes=[pltpu.VMEM((1, 64), jnp.float32)],
    )(vec, wv1, bv1, wv2, bv2)


def kernel(observation, action, edge_index, We1, be1, We2, be2, Wg, bg,
           Wd, bd, Wp1, bp1, Wp2, bp2, Wv1, bv1, Wv2, bv2):
    obs = observation.reshape(N, -1)
    act = action.reshape(N, -1)
    ei = edge_index.astype(jnp.int32)
    pad = jnp.full((NW, EPW_PAD - EPW), DUMMY, jnp.int32)
    src_g = jnp.concatenate([ei[0].reshape(NW, EPW), pad], 1).reshape(NW, NG, CH)
    dst_g = jnp.concatenate([ei[1].reshape(NW, EPW), pad], 1).reshape(NW, NG, CH)

    ones128 = jnp.ones((CH,), jnp.float32)
    zeros_n = jnp.zeros((NPAD,), jnp.float32)
    zeros_rows = jnp.zeros((RPS, HID), jnp.float32)

    _deg, _agg = _sc_kernels()
    deg2 = _deg(dst_g, ones128, zeros_n)
    degT = jnp.transpose(deg2[:, :N])
    X, hp, dinv = _enc(obs, act, degT, We1[:128], We1[128:],
                       be1.reshape(1, -1), We2, be2.reshape(1, -1), Wg)
    hp_pad = jnp.concatenate([hp, jnp.zeros((NPAD - N, HID), jnp.float32)], 0)

    agg2 = _agg(src_g, dst_g, hp_pad, zeros_rows)

    xp2 = _post(agg2[0, :N], agg2[1, :N], hp, dinv, bg.reshape(1, -1),
                Wd, bd.reshape(1, -1), X, Wp1[:HID], Wp1[HID:],
                bp1.reshape(1, -1), Wp2, bp2.reshape(1, -1))
    vec = xp2.reshape(1, N * HID)
    out = _value(vec, Wv1, bv1.reshape(1, -1), Wv2, bv2.reshape(1, 1))
    return out.reshape(1)


# speedup vs baseline: 1.2632x; 1.2627x over previous
import functools

import jax
import jax.numpy as jnp
from jax import lax
from jax.experimental import pallas as pl
from jax.experimental.pallas import tpu as pltpu
from jax.experimental.pallas import tpu_sc as plsc

N = 10000
E = 320000
HID = 32
NW = 32
EPW = E // NW
CH = 128
NG = 79
EPW_PAD = NG * CH
NPAD = N + 112
DUMMY = N
RPS = NPAD // 16

def _deg_body(dst_g, ones_hbm, zeros_hbm, deg_out, idx_v, ones_v, zstage_v, acc):
    cid = lax.axis_index("c")
    sid = lax.axis_index("s")
    wid = sid * 2 + cid
    pltpu.sync_copy(dst_g.at[wid], idx_v)
    pltpu.sync_copy(ones_hbm, ones_v)

    @pl.when(sid == 0)
    def _():
        pltpu.sync_copy(zeros_hbm, zstage_v)
        pltpu.sync_copy(zstage_v, acc)

    plsc.subcore_barrier()

    @pl.loop(0, NG)
    def _(j):
        pltpu.sync_copy(ones_v, acc.at[idx_v.at[j]], add=True)

    plsc.subcore_barrier()

    @pl.when(sid == 0)
    def _():
        pltpu.sync_copy(acc, deg_out.at[cid])


def _agg_body(src_g, dst_g, hp_hbm, zeros_hbm, agg_out,
              sidx_v, didx_v, rows0_v, rows1_v, zbuf_v, acc, sem0, sem1):
    cid = lax.axis_index("c")
    sid = lax.axis_index("s")
    wid = sid * 2 + cid
    pltpu.sync_copy(src_g.at[wid], sidx_v)
    pltpu.sync_copy(dst_g.at[wid], didx_v)
    pltpu.sync_copy(zeros_hbm, zbuf_v)
    pltpu.sync_copy(zbuf_v, acc.at[pl.ds(sid * RPS, RPS)])
    plsc.subcore_barrier()

    @pl.loop(0, NG)
    def _(j):
        pltpu.async_copy(hp_hbm.at[sidx_v.at[j]], rows0_v, sem0).wait()
        pltpu.sync_copy(rows0_v, acc.at[didx_v.at[j]], add=True)

    plsc.subcore_barrier()
    pltpu.sync_copy(acc.at[pl.ds(sid * RPS, RPS)],
                    agg_out.at[cid, pl.ds(sid * RPS, RPS)])


@functools.cache
def _sc_kernels():
    mesh = plsc.VectorSubcoreMesh(core_axis_name="c", subcore_axis_name="s",
                                  num_cores=2, num_subcores=16)
    params = pltpu.CompilerParams(use_tc_tiling_on_sc=False)
    deg = pl.kernel(
        _deg_body,
        out_type=jax.ShapeDtypeStruct((2, NPAD), jnp.float32),
        mesh=mesh,
        compiler_params=params,
        scratch_types=[
            pltpu.VMEM((NG, CH), jnp.int32),
            pltpu.VMEM((CH,), jnp.float32),
            pltpu.VMEM((NPAD,), jnp.float32),
            pltpu.VMEM_SHARED((NPAD,), jnp.float32),
        ],
    )
    agg = pl.kernel(
        _agg_body,
        out_type=jax.ShapeDtypeStruct((2, NPAD, HID), jnp.float32),
        mesh=mesh,
        compiler_params=params,
        scratch_types=[
            pltpu.VMEM((NG, CH), jnp.int32),
            pltpu.VMEM((NG, CH), jnp.int32),
            pltpu.VMEM((CH, HID), jnp.float32),
            pltpu.VMEM((CH, HID), jnp.float32),
            pltpu.VMEM((RPS, HID), jnp.float32),
            pltpu.VMEM_SHARED((NPAD, HID), jnp.float32),
            pltpu.SemaphoreType.DMA,
            pltpu.SemaphoreType.DMA,
        ],
    )
    return deg, agg


def _enc_body(obs_ref, act_ref, degT_ref, w1a, w1b, b1, w2, b2, wg,
              x_out, hp_out, dinv_out):
    f32 = jnp.float32
    x1 = jnp.dot(obs_ref[...], w1a[...], preferred_element_type=f32)
    x1 = x1 + jnp.dot(act_ref[...], w1b[...], preferred_element_type=f32)
    x1 = jnp.maximum(x1 + b1[...], 0.0)
    x = jnp.maximum(jnp.dot(x1, w2[...], preferred_element_type=f32) + b2[...], 0.0)
    x_out[...] = x
    h = jnp.dot(x, wg[...], preferred_element_type=f32)
    d = degT_ref[...]
    deg = d[:, 0:1] + d[:, 1:2] + 1.0
    dinv = lax.rsqrt(deg)
    dinv_out[...] = dinv
    hp_out[...] = h * dinv


def _enc(obs, act, degT, w1a, w1b, b1, w2, b2, wg):
    return pl.pallas_call(
        _enc_body,
        out_shape=[
            jax.ShapeDtypeStruct((N, HID), jnp.float32),
            jax.ShapeDtypeStruct((N, HID), jnp.float32),
            jax.ShapeDtypeStruct((N, 1), jnp.float32),
        ],
    )(obs, act, degT, w1a, w1b, b1, w2, b2, wg)


def _post_body(a0, a1, hp, dinv, bg, wd, bd, x, wp1a, wp1b, bp1, wp2, bp2, out):
    f32 = jnp.float32
    agg = a0[...] + a1[...] + hp[...]
    xg = jnp.maximum(agg * dinv[...] + bg[...], 0.0)
    xg = jnp.maximum(jnp.dot(xg, wd[...], preferred_element_type=f32) + bd[...], 0.0)
    xp = jnp.dot(xg, wp1a[...], preferred_element_type=f32)
    xp = xp + jnp.dot(x[...], wp1b[...], preferred_element_type=f32)
    xp = jnp.maximum(xp + bp1[...], 0.0)
    xp = jnp.maximum(jnp.dot(xp, wp2[...], preferred_element_type=f32) + bp2[...], 0.0)
    out[...] = xp


def _post(a0, a1, hp, dinv, bg, wd, bd, x, wp1a, wp1b, bp1, wp2, bp2):
    return pl.pallas_call(
        _post_body,
        out_shape=jax.ShapeDtypeStruct((N, HID), jnp.float32),
    )(a0, a1, hp, dinv, bg, wd, bd, x, wp1a, wp1b, bp1, wp2, bp2)


_VBLK = 32000
_VGRID = (N * HID) // _VBLK


def _value_body(vec_ref, wv1_ref, bv1_ref, wv2_ref, bv2_ref, out_ref, acc_ref):
    k = pl.program_id(0)

    @pl.when(k == 0)
    def _():
        acc_ref[...] = jnp.zeros_like(acc_ref)

    acc_ref[...] += jnp.dot(vec_ref[...], wv1_ref[...],
                            preferred_element_type=jnp.float32)

    @pl.when(k == _VGRID - 1)
    def _():
        v = jnp.maximum(acc_ref[...] + bv1_ref[...], 0.0)
        out_ref[...] = jnp.dot(v, wv2_ref[...],
                               preferred_element_type=jnp.float32) + bv2_ref[...]


def _value(vec, wv1, bv1, wv2, bv2):
    return pl.pallas_call(
        _value_body,
        grid=(_VGRID,),
        in_specs=[
            pl.BlockSpec((1, _VBLK), lambda k: (0, k)),
            pl.BlockSpec((_VBLK, 64), lambda k: (k, 0)),
            pl.BlockSpec((1, 64), lambda k: (0, 0)),
            pl.BlockSpec((64, 1), lambda k: (0, 0)),
            pl.BlockSpec((1, 1), lambda k: (0, 0)),
        ],
        out_specs=pl.BlockSpec((1, 1), lambda k: (0, 0)),
        out_shape=jax.ShapeDtypeStruct((1, 1), jnp.float32),
        scratch_shapes=[pltpu.VMEM((1, 64), jnp.float32)],
    )(vec, wv1, bv1, wv2, bv2)


def kernel(observation, action, edge_index, We1, be1, We2, be2, Wg, bg,
           Wd, bd, Wp1, bp1, Wp2, bp2, Wv1, bv1, Wv2, bv2):
    obs = observation.reshape(N, -1)
    act = action.reshape(N, -1)
    ei = edge_index.astype(jnp.int32)
    pad = jnp.broadcast_to(
        DUMMY + jnp.arange(EPW_PAD - EPW, dtype=jnp.int32) % (NPAD - N),
        (NW, EPW_PAD - EPW))
    src_g = jnp.concatenate([ei[0].reshape(NW, EPW), pad], 1).reshape(NW, NG, CH)
    dst_g = jnp.concatenate([ei[1].reshape(NW, EPW), pad], 1).reshape(NW, NG, CH)

    ones128 = jnp.ones((CH,), jnp.float32)
    zeros_n = jnp.zeros((NPAD,), jnp.float32)
    zeros_rows = jnp.zeros((RPS, HID), jnp.float32)

    _deg, _agg = _sc_kernels()
    deg2 = _deg(dst_g, ones128, zeros_n)
    degT = jnp.transpose(deg2[:, :N])
    X, hp, dinv = _enc(obs, act, degT, We1[:128], We1[128:],
                       be1.reshape(1, -1), We2, be2.reshape(1, -1), Wg)
    hp_pad = jnp.concatenate([hp, jnp.zeros((NPAD - N, HID), jnp.float32)], 0)

    agg2 = _agg(src_g, dst_g, hp_pad, zeros_rows)

    xp2 = _post(agg2[0, :N], agg2[1, :N], hp, dinv, bg.reshape(1, -1),
                Wd, bd.reshape(1, -1), X, Wp1[:HID], Wp1[HID:],
                bp1.reshape(1, -1), Wp2, bp2.reshape(1, -1))
    vec = xp2.reshape(1, N * HID)
    out = _value(vec, Wv1, bv1.reshape(1, -1), Wv2, bv2.reshape(1, 1))
    return out.reshape(1)


# speedup vs baseline: 1.2695x; 1.0050x over previous
import functools

import jax
import jax.numpy as jnp
from jax import lax
from jax.experimental import pallas as pl
from jax.experimental.pallas import tpu as pltpu
from jax.experimental.pallas import tpu_sc as plsc

N = 10000
E = 320000
HID = 32
NW = 32
EPW = E // NW
CH = 128
NG = 79
EPW_PAD = NG * CH
NPAD = N + 112
DUMMY = N
RPS = NPAD // 16

def _deg_body(dst_g, ones_hbm, zeros_hbm, deg_out, idx_v, ones_v, zstage_v, acc):
    cid = lax.axis_index("c")
    sid = lax.axis_index("s")
    wid = sid * 2 + cid
    pltpu.sync_copy(dst_g.at[wid], idx_v)
    pltpu.sync_copy(ones_hbm, ones_v)

    @pl.when(sid == 0)
    def _():
        pltpu.sync_copy(zeros_hbm, zstage_v)
        pltpu.sync_copy(zstage_v, acc)

    plsc.subcore_barrier()

    @pl.loop(0, NG)
    def _(j):
        pltpu.sync_copy(ones_v, acc.at[idx_v.at[j]], add=True)

    plsc.subcore_barrier()

    @pl.when(sid == 0)
    def _():
        pltpu.sync_copy(acc, deg_out.at[cid])


def _agg_body(src_g, dst_g, hp_hbm, zeros_hbm, agg_out,
              sidx_v, didx_v, rows0_v, rows1_v, zbuf_v, acc, sem0, sem1):
    cid = lax.axis_index("c")
    sid = lax.axis_index("s")
    wid = sid * 2 + cid
    pltpu.sync_copy(src_g.at[wid], sidx_v)
    pltpu.sync_copy(dst_g.at[wid], didx_v)
    pltpu.sync_copy(zeros_hbm, zbuf_v)
    pltpu.sync_copy(zbuf_v, acc.at[pl.ds(sid * RPS, RPS)])
    plsc.subcore_barrier()

    pltpu.async_copy(hp_hbm.at[sidx_v.at[0]], rows0_v, sem0)

    @pl.loop(0, NG - 1)
    def _(j):
        odd = lax.rem(j, 2)

        @pl.when(odd == 0)
        def _():
            pltpu.async_copy(hp_hbm.at[sidx_v.at[j + 1]], rows1_v, sem1)
            pltpu.make_async_copy(hp_hbm.at[sidx_v.at[j]], rows0_v, sem0).wait()
            pltpu.sync_copy(rows0_v, acc.at[didx_v.at[j]], add=True)

        @pl.when(odd == 1)
        def _():
            pltpu.async_copy(hp_hbm.at[sidx_v.at[j + 1]], rows0_v, sem0)
            pltpu.make_async_copy(hp_hbm.at[sidx_v.at[j]], rows1_v, sem1).wait()
            pltpu.sync_copy(rows1_v, acc.at[didx_v.at[j]], add=True)

    pltpu.make_async_copy(hp_hbm.at[sidx_v.at[NG - 1]], rows0_v, sem0).wait()
    pltpu.sync_copy(rows0_v, acc.at[didx_v.at[NG - 1]], add=True)

    plsc.subcore_barrier()
    pltpu.sync_copy(acc.at[pl.ds(sid * RPS, RPS)],
                    agg_out.at[cid, pl.ds(sid * RPS, RPS)])


@functools.cache
def _sc_kernels():
    mesh = plsc.VectorSubcoreMesh(core_axis_name="c", subcore_axis_name="s",
                                  num_cores=2, num_subcores=16)
    params = pltpu.CompilerParams(use_tc_tiling_on_sc=False)
    deg = pl.kernel(
        _deg_body,
        out_type=jax.ShapeDtypeStruct((2, NPAD), jnp.float32),
        mesh=mesh,
        compiler_params=params,
        scratch_types=[
            pltpu.VMEM((NG, CH), jnp.int32),
            pltpu.VMEM((CH,), jnp.float32),
            pltpu.VMEM((NPAD,), jnp.float32),
            pltpu.VMEM_SHARED((NPAD,), jnp.float32),
        ],
    )
    agg = pl.kernel(
        _agg_body,
        out_type=jax.ShapeDtypeStruct((2, NPAD, HID), jnp.float32),
        mesh=mesh,
        compiler_params=params,
        scratch_types=[
            pltpu.VMEM((NG, CH), jnp.int32),
            pltpu.VMEM((NG, CH), jnp.int32),
            pltpu.VMEM((CH, HID), jnp.float32),
            pltpu.VMEM((CH, HID), jnp.float32),
            pltpu.VMEM((RPS, HID), jnp.float32),
            pltpu.VMEM_SHARED((NPAD, HID), jnp.float32),
            pltpu.SemaphoreType.DMA,
            pltpu.SemaphoreType.DMA,
        ],
    )
    return deg, agg


def _enc_body(obs_ref, act_ref, degT_ref, w1a, w1b, b1, w2, b2, wg,
              x_out, hp_out, dinv_out):
    f32 = jnp.float32
    x1 = jnp.dot(obs_ref[...], w1a[...], preferred_element_type=f32)
    x1 = x1 + jnp.dot(act_ref[...], w1b[...], preferred_element_type=f32)
    x1 = jnp.maximum(x1 + b1[...], 0.0)
    x = jnp.maximum(jnp.dot(x1, w2[...], preferred_element_type=f32) + b2[...], 0.0)
    x_out[...] = x
    h = jnp.dot(x, wg[...], preferred_element_type=f32)
    d = degT_ref[...]
    deg = d[:, 0:1] + d[:, 1:2] + 1.0
    dinv = lax.rsqrt(deg)
    dinv_out[...] = dinv
    hp_out[...] = h * dinv


def _enc(obs, act, degT, w1a, w1b, b1, w2, b2, wg):
    return pl.pallas_call(
        _enc_body,
        out_shape=[
            jax.ShapeDtypeStruct((N, HID), jnp.float32),
            jax.ShapeDtypeStruct((N, HID), jnp.float32),
            jax.ShapeDtypeStruct((N, 1), jnp.float32),
        ],
    )(obs, act, degT, w1a, w1b, b1, w2, b2, wg)


def _post_body(a0, a1, hp, dinv, bg, wd, bd, x, wp1a, wp1b, bp1, wp2, bp2, out):
    f32 = jnp.float32
    agg = a0[...] + a1[...] + hp[...]
    xg = jnp.maximum(agg * dinv[...] + bg[...], 0.0)
    xg = jnp.maximum(jnp.dot(xg, wd[...], preferred_element_type=f32) + bd[...], 0.0)
    xp = jnp.dot(xg, wp1a[...], preferred_element_type=f32)
    xp = xp + jnp.dot(x[...], wp1b[...], preferred_element_type=f32)
    xp = jnp.maximum(xp + bp1[...], 0.0)
    xp = jnp.maximum(jnp.dot(xp, wp2[...], preferred_element_type=f32) + bp2[...], 0.0)
    out[...] = xp


def _post(a0, a1, hp, dinv, bg, wd, bd, x, wp1a, wp1b, bp1, wp2, bp2):
    return pl.pallas_call(
        _post_body,
        out_shape=jax.ShapeDtypeStruct((N, HID), jnp.float32),
    )(a0, a1, hp, dinv, bg, wd, bd, x, wp1a, wp1b, bp1, wp2, bp2)


_VBLK = 32000
_VGRID = (N * HID) // _VBLK


def _value_body(vec_ref, wv1_ref, bv1_ref, wv2_ref, bv2_ref, out_ref, acc_ref):
    k = pl.program_id(0)

    @pl.when(k == 0)
    def _():
        acc_ref[...] = jnp.zeros_like(acc_ref)

    acc_ref[...] += jnp.dot(vec_ref[...], wv1_ref[...],
                            preferred_element_type=jnp.float32)

    @pl.when(k == _VGRID - 1)
    def _():
        v = jnp.maximum(acc_ref[...] + bv1_ref[...], 0.0)
        out_ref[...] = jnp.dot(v, wv2_ref[...],
                               preferred_element_type=jnp.float32) + bv2_ref[...]


def _value(vec, wv1, bv1, wv2, bv2):
    return pl.pallas_call(
        _value_body,
        grid=(_VGRID,),
        in_specs=[
            pl.BlockSpec((1, _VBLK), lambda k: (0, k)),
            pl.BlockSpec((_VBLK, 64), lambda k: (k, 0)),
            pl.BlockSpec((1, 64), lambda k: (0, 0)),
            pl.BlockSpec((64, 1), lambda k: (0, 0)),
            pl.BlockSpec((1, 1), lambda k: (0, 0)),
        ],
        out_specs=pl.BlockSpec((1, 1), lambda k: (0, 0)),
        out_shape=jax.ShapeDtypeStruct((1, 1), jnp.float32),
        scratch_shapes=[pltpu.VMEM((1, 64), jnp.float32)],
    )(vec, wv1, bv1, wv2, bv2)


def kernel(observation, action, edge_index, We1, be1, We2, be2, Wg, bg,
           Wd, bd, Wp1, bp1, Wp2, bp2, Wv1, bv1, Wv2, bv2):
    obs = observation.reshape(N, -1)
    act = action.reshape(N, -1)
    ei = edge_index.astype(jnp.int32)
    pad = jnp.broadcast_to(
        DUMMY + jnp.arange(EPW_PAD - EPW, dtype=jnp.int32) % (NPAD - N),
        (NW, EPW_PAD - EPW))
    src_g = jnp.concatenate([ei[0].reshape(NW, EPW), pad], 1).reshape(NW, NG, CH)
    dst_g = jnp.concatenate([ei[1].reshape(NW, EPW), pad], 1).reshape(NW, NG, CH)

    ones128 = jnp.ones((CH,), jnp.float32)
    zeros_n = jnp.zeros((NPAD,), jnp.float32)
    zeros_rows = jnp.zeros((RPS, HID), jnp.float32)

    _deg, _agg = _sc_kernels()
    deg2 = _deg(dst_g, ones128, zeros_n)
    degT = jnp.transpose(deg2[:, :N])
    X, hp, dinv = _enc(obs, act, degT, We1[:128], We1[128:],
                       be1.reshape(1, -1), We2, be2.reshape(1, -1), Wg)
    hp_pad = jnp.concatenate([hp, jnp.zeros((NPAD - N, HID), jnp.float32)], 0)

    agg2 = _agg(src_g, dst_g, hp_pad, zeros_rows)

    xp2 = _post(agg2[0, :N], agg2[1, :N], hp, dinv, bg.reshape(1, -1),
                Wd, bd.reshape(1, -1), X, Wp1[:HID], Wp1[HID:],
                bp1.reshape(1, -1), Wp2, bp2.reshape(1, -1))
    vec = xp2.reshape(1, N * HID)
    out = _value(vec, Wv1, bv1.reshape(1, -1), Wv2, bv2.reshape(1, 1))
    return out.reshape(1)


# speedup vs baseline: 1.8862x; 1.4858x over previous
import functools

import jax
import jax.numpy as jnp
from jax import lax
from jax.experimental import pallas as pl
from jax.experimental.pallas import tpu as pltpu
from jax.experimental.pallas import tpu_sc as plsc

N = 10000
E = 320000
HID = 32
NW = 32
EPW = E // NW
CH = 128
NG = 79
EPW_PAD = NG * CH
NPAD = N + 112
DUMMY = N
RPS = NPAD // 16

def _deg_body(dst_g, ones_hbm, zeros_hbm, deg_out, idx_v, ones_v, zstage_v, acc):
    cid = lax.axis_index("c")
    sid = lax.axis_index("s")
    wid = sid * 2 + cid
    pltpu.sync_copy(dst_g.at[wid], idx_v)
    pltpu.sync_copy(ones_hbm, ones_v)

    @pl.when(sid == 0)
    def _():
        pltpu.sync_copy(zeros_hbm, zstage_v)
        pltpu.sync_copy(zstage_v, acc)

    plsc.subcore_barrier()

    @pl.loop(0, NG)
    def _(j):
        pltpu.sync_copy(ones_v, acc.at[idx_v.at[j]], add=True)

    plsc.subcore_barrier()

    @pl.when(sid == 0)
    def _():
        pltpu.sync_copy(acc, deg_out.at[cid])


def _agg_body(src_g, dst_g, hp_hbm, zeros_hbm, agg_out,
              sidx_v, didx_v, rows0_v, rows1_v, zbuf_v, acc, sem0, sem1):
    cid = lax.axis_index("c")
    sid = lax.axis_index("s")
    wid = sid * 2 + cid
    pltpu.sync_copy(src_g.at[wid], sidx_v)
    pltpu.sync_copy(dst_g.at[wid], didx_v)
    pltpu.sync_copy(zeros_hbm, zbuf_v)
    pltpu.sync_copy(zbuf_v, acc.at[pl.ds(sid * RPS, RPS)])
    plsc.subcore_barrier()

    pltpu.async_copy(hp_hbm.at[sidx_v.at[0]], rows0_v, sem0)

    @pl.loop(0, NG - 1)
    def _(j):
        odd = lax.rem(j, 2)

        @pl.when(odd == 0)
        def _():
            pltpu.async_copy(hp_hbm.at[sidx_v.at[j + 1]], rows1_v, sem1)
            pltpu.make_async_copy(hp_hbm.at[sidx_v.at[j]], rows0_v, sem0).wait()
            pltpu.sync_copy(rows0_v, acc.at[didx_v.at[j]], add=True)

        @pl.when(odd == 1)
        def _():
            pltpu.async_copy(hp_hbm.at[sidx_v.at[j + 1]], rows0_v, sem0)
            pltpu.make_async_copy(hp_hbm.at[sidx_v.at[j]], rows1_v, sem1).wait()
            pltpu.sync_copy(rows1_v, acc.at[didx_v.at[j]], add=True)

    pltpu.make_async_copy(hp_hbm.at[sidx_v.at[NG - 1]], rows0_v, sem0).wait()
    pltpu.sync_copy(rows0_v, acc.at[didx_v.at[NG - 1]], add=True)

    plsc.subcore_barrier()
    pltpu.sync_copy(acc.at[pl.ds(sid * RPS, RPS)],
                    agg_out.at[cid, pl.ds(sid * RPS, RPS)])


@functools.cache
def _sc_kernels():
    mesh = plsc.VectorSubcoreMesh(core_axis_name="c", subcore_axis_name="s",
                                  num_cores=2, num_subcores=16)
    params = pltpu.CompilerParams(use_tc_tiling_on_sc=False)
    deg = pl.kernel(
        _deg_body,
        out_type=jax.ShapeDtypeStruct((2, NPAD), jnp.float32),
        mesh=mesh,
        compiler_params=params,
        scratch_types=[
            pltpu.VMEM((NG, CH), jnp.int32),
            pltpu.VMEM((CH,), jnp.float32),
            pltpu.VMEM((NPAD,), jnp.float32),
            pltpu.VMEM_SHARED((NPAD,), jnp.float32),
        ],
    )
    agg = pl.kernel(
        _agg_body,
        out_type=jax.ShapeDtypeStruct((2, NPAD, HID), jnp.float32),
        mesh=mesh,
        compiler_params=params,
        scratch_types=[
            pltpu.VMEM((NG, CH), jnp.int32),
            pltpu.VMEM((NG, CH), jnp.int32),
            pltpu.VMEM((CH, HID), jnp.float32),
            pltpu.VMEM((CH, HID), jnp.float32),
            pltpu.VMEM((RPS, HID), jnp.float32),
            pltpu.VMEM_SHARED((NPAD, HID), jnp.float32),
            pltpu.SemaphoreType.DMA,
            pltpu.SemaphoreType.DMA,
        ],
    )
    return deg, agg


def _enc_body(obs_ref, act_ref, degT_ref, w1a, w1b, b1, w2, b2, wg,
              x_out, hp_out, dinv_out):
    f32 = jnp.float32
    x1 = jnp.dot(obs_ref[...], w1a[...], preferred_element_type=f32)
    x1 = x1 + jnp.dot(act_ref[...], w1b[...], preferred_element_type=f32)
    x1 = jnp.maximum(x1 + b1[...], 0.0)
    x = jnp.maximum(jnp.dot(x1, w2[...], preferred_element_type=f32) + b2[...], 0.0)
    x_out[...] = x
    h = jnp.dot(x, wg[...], preferred_element_type=f32)
    d = degT_ref[...]
    deg = d[:, 0:1] + d[:, 1:2] + 1.0
    dinv = lax.rsqrt(deg)
    dinv_out[...] = dinv
    hp_out[...] = h * dinv


def _enc(obs, act, degT, w1a, w1b, b1, w2, b2, wg):
    return pl.pallas_call(
        _enc_body,
        out_shape=[
            jax.ShapeDtypeStruct((N, HID), jnp.float32),
            jax.ShapeDtypeStruct((N, HID), jnp.float32),
            jax.ShapeDtypeStruct((N, 1), jnp.float32),
        ],
    )(obs, act, degT, w1a, w1b, b1, w2, b2, wg)


def _post_body(a0, a1, hp, dinv, bg, wd, bd, x, wp1a, wp1b, bp1, wp2, bp2, out):
    f32 = jnp.float32
    agg = a0[...] + a1[...] + hp[...]
    xg = jnp.maximum(agg * dinv[...] + bg[...], 0.0)
    xg = jnp.maximum(jnp.dot(xg, wd[...], preferred_element_type=f32) + bd[...], 0.0)
    xp = jnp.dot(xg, wp1a[...], preferred_element_type=f32)
    xp = xp + jnp.dot(x[...], wp1b[...], preferred_element_type=f32)
    xp = jnp.maximum(xp + bp1[...], 0.0)
    xp = jnp.maximum(jnp.dot(xp, wp2[...], preferred_element_type=f32) + bp2[...], 0.0)
    out[...] = xp


def _post(a0, a1, hp, dinv, bg, wd, bd, x, wp1a, wp1b, bp1, wp2, bp2):
    return pl.pallas_call(
        _post_body,
        out_shape=jax.ShapeDtypeStruct((N, HID), jnp.float32),
    )(a0, a1, hp, dinv, bg, wd, bd, x, wp1a, wp1b, bp1, wp2, bp2)


_VBLK = 32000
_VGRID = (N * HID) // _VBLK


def _value_body(vec_ref, wv1t_ref, bv1_ref, wv2_ref, bv2_ref, out_ref, acc_ref):
    k = pl.program_id(0)

    @pl.when(k == 0)
    def _():
        acc_ref[...] = jnp.zeros_like(acc_ref)

    acc_ref[...] += lax.dot_general(
        vec_ref[...], wv1t_ref[...],
        dimension_numbers=(((1,), (1,)), ((), ())),
        preferred_element_type=jnp.float32)

    @pl.when(k == _VGRID - 1)
    def _():
        v = jnp.maximum(acc_ref[...] + bv1_ref[...], 0.0)
        out_ref[...] = jnp.dot(v, wv2_ref[...],
                               preferred_element_type=jnp.float32) + bv2_ref[...]


def _value(vec, wv1t, bv1, wv2, bv2):
    return pl.pallas_call(
        _value_body,
        grid=(_VGRID,),
        in_specs=[
            pl.BlockSpec((1, _VBLK), lambda k: (0, k)),
            pl.BlockSpec((64, _VBLK), lambda k: (0, k)),
            pl.BlockSpec((1, 64), lambda k: (0, 0)),
            pl.BlockSpec((64, 1), lambda k: (0, 0)),
            pl.BlockSpec((1, 1), lambda k: (0, 0)),
        ],
        out_specs=pl.BlockSpec((1, 1), lambda k: (0, 0)),
        out_shape=jax.ShapeDtypeStruct((1, 1), jnp.float32),
        scratch_shapes=[pltpu.VMEM((1, 64), jnp.float32)],
    )(vec, wv1t, bv1, wv2, bv2)


def kernel(observation, action, edge_index, We1, be1, We2, be2, Wg, bg,
           Wd, bd, Wp1, bp1, Wp2, bp2, Wv1, bv1, Wv2, bv2):
    obs = observation.reshape(N, -1)
    act = action.reshape(N, -1)
    ei = edge_index.astype(jnp.int32)
    pad = jnp.broadcast_to(
        DUMMY + jnp.arange(EPW_PAD - EPW, dtype=jnp.int32) % (NPAD - N),
        (NW, EPW_PAD - EPW))
    src_g = jnp.concatenate([ei[0].reshape(NW, EPW), pad], 1).reshape(NW, NG, CH)
    dst_g = jnp.concatenate([ei[1].reshape(NW, EPW), pad], 1).reshape(NW, NG, CH)

    ones128 = jnp.ones((CH,), jnp.float32)
    zeros_n = jnp.zeros((NPAD,), jnp.float32)
    zeros_rows = jnp.zeros((RPS, HID), jnp.float32)

    _deg, _agg = _sc_kernels()
    deg2 = _deg(dst_g, ones128, zeros_n)
    degT = jnp.transpose(deg2[:, :N])
    X, hp, dinv = _enc(obs, act, degT, We1[:128], We1[128:],
                       be1.reshape(1, -1), We2, be2.reshape(1, -1), Wg)
    hp_pad = jnp.concatenate([hp, jnp.zeros((NPAD - N, HID), jnp.float32)], 0)

    agg2 = _agg(src_g, dst_g, hp_pad, zeros_rows)

    xp2 = _post(agg2[0, :N], agg2[1, :N], hp, dinv, bg.reshape(1, -1),
                Wd, bd.reshape(1, -1), X, Wp1[:HID], Wp1[HID:],
                bp1.reshape(1, -1), Wp2, bp2.reshape(1, -1))
    vec = xp2.reshape(1, N * HID)
    out = _value(vec, Wv1.T, bv1.reshape(1, -1), Wv2, bv2.reshape(1, 1))
    return out.reshape(1)
